# Initial kernel scaffold; baseline (speedup 1.0000x reference)
import jax
import jax.numpy as jnp
from jax.experimental import pallas as pl


def kernel(channel_items, table):
    raise NotImplementedError("write your pallas kernel here")


# speedup vs baseline: 1.9276x; 1.9276x over previous
import functools

import jax
import jax.numpy as jnp
from jax import lax
from jax.experimental import pallas as pl
from jax.experimental.pallas import tpu as pltpu
from jax.experimental.pallas import tpu_sc as plsc

D = 64
K = 10
NC = 2
NS = 16
NW = NC * NS
CHUNK_OUT = 8
CHUNK_IDX = CHUNK_OUT * K
LANES = 16
DV = D // LANES


@functools.cache
def _make_kernel(n_out: int):
    per_w = n_out // NW
    nchunk = per_w // CHUNK_OUT
    assert per_w * NW == n_out and nchunk * CHUNK_OUT == per_w
    assert nchunk % 2 == 0
    mesh = plsc.VectorSubcoreMesh(core_axis_name="c", subcore_axis_name="s")

    @functools.partial(
        pl.kernel,
        mesh=mesh,
        compiler_params=pltpu.CompilerParams(use_tc_tiling_on_sc=False),
        out_type=jax.ShapeDtypeStruct((n_out, D), jnp.float32),
        scratch_types=[
            pltpu.VMEM((nchunk, CHUNK_IDX), jnp.int32),
            pltpu.VMEM((CHUNK_IDX, D), jnp.float32),
            pltpu.VMEM((CHUNK_IDX, D), jnp.float32),
            pltpu.VMEM((CHUNK_OUT, D), jnp.float32),
            pltpu.SemaphoreType.DMA,
            pltpu.SemaphoreType.DMA,
        ],
    )
    def k(idx_hbm, table_hbm, out_hbm, idx_v, rows0, rows1, out_v, sem0, sem1):
        wid = lax.axis_index("s") * NC + lax.axis_index("c")
        rows = (rows0, rows1)
        sems = (sem0, sem1)
        pltpu.sync_copy(idx_hbm.at[wid], idx_v)
        pltpu.async_copy(table_hbm.at[idx_v.at[0]], rows0, sem0)

        def outer(i, carry):
            for b in range(2):
                c = i * 2 + b
                nb = 1 - b
                cn = jnp.minimum(c + 1, nchunk - 1)
                pltpu.async_copy(table_hbm.at[idx_v.at[cn]], rows[nb], sems[nb])
                pltpu.make_async_copy(
                    table_hbm.at[idx_v.at[c]], rows[b], sems[b]
                ).wait()
                r = rows[b]
                for o in range(CHUNK_OUT):
                    base = o * K
                    for d in range(DV):
                        sl = pl.ds(d * LANES, LANES)
                        acc = r[base, sl]
                        for kk in range(1, K):
                            acc = acc + r[base + kk, sl]
                        out_v[o, sl] = acc * jnp.float32(1.0 / K)
                pltpu.sync_copy(
                    out_v,
                    out_hbm.at[pl.ds(wid * per_w + c * CHUNK_OUT, CHUNK_OUT)],
                )
            return carry

        lax.fori_loop(0, nchunk // 2, outer, 0)
        pltpu.make_async_copy(table_hbm.at[idx_v.at[0]], rows0, sem0).wait()

    return k


def kernel(channel_items, table):
    B, C, Kk = channel_items.shape
    n_out = B * C
    idx = channel_items.astype(jnp.int32).reshape(
        NW, n_out * Kk // (NW * CHUNK_IDX), CHUNK_IDX
    )
    out = _make_kernel(n_out)(idx, table)
    return out.reshape(B, C, D)
